# baseline (speedup 1.0000x reference)
import functools

import jax
import jax.numpy as jnp
from jax import lax
from jax.experimental import pallas as pl
from jax.experimental.pallas import tpu as pltpu
from jax.experimental.pallas import tpu_sc as plsc

N_GRID = 50
NSEG = N_GRID - 1
TAB = 64
NC, NS, L = 2, 16, 16
NW = NC * NS

ROWS, COLS = 4096, 2048
ROWS_W = ROWS // NW
CROWS = 16
CHUNK = CROWS * COLS
NCHUNK = ROWS_W // CROWS
RVECS = COLS // L
COLS_LOG2 = 11
NBUF = 3

INV_H = NSEG / 10.0
OFF = 5.0 * INV_H

_mesh = plsc.VectorSubcoreMesh(
    core_axis_name="c", subcore_axis_name="s", num_cores=NC, num_subcores=NS
)


@functools.partial(
    pl.kernel,
    out_type=jax.ShapeDtypeStruct((ROWS, COLS), jnp.float32),
    mesh=_mesh,
    compiler_params=pltpu.CompilerParams(needs_layout_passes=False),
    scratch_types=[
        pltpu.VMEM((N_GRID,), jnp.float32),
        pltpu.VMEM((N_GRID,), jnp.float32),
        pltpu.VMEM((TAB,), jnp.float32),
        pltpu.VMEM((TAB,), jnp.float32),
        pltpu.VMEM((NBUF, CROWS, COLS), jnp.float32),
        pltpu.SemaphoreType.DMA((NBUF,)),
        pltpu.SemaphoreType.DMA((NBUF,)),
    ],
)
def _interp_sc(x_hbm, y_hbm, g_hbm, out_hbm, ybuf, gbuf, tab_s, tab_a,
               ring, isems, osems):
    wid = lax.axis_index("s") * NC + lax.axis_index("c")

    pltpu.sync_copy(y_hbm, ybuf)
    pltpu.sync_copy(g_hbm, gbuf)

    lanes = lax.iota(jnp.int32, L)
    for j in range(TAB // L):
        k0 = jnp.minimum(lanes + (j * L), NSEG - 1)
        k1 = k0 + 1
        y0 = plsc.load_gather(ybuf, [k0])
        y1 = plsc.load_gather(ybuf, [k1])
        g0 = plsc.load_gather(gbuf, [k0])
        g1 = plsc.load_gather(gbuf, [k1])
        s = (y1 - y0) / (g1 - g0)
        tab_s[pl.ds(j * L, L)] = s
        tab_a[pl.ds(j * L, L)] = y0 - s * g0

    base = wid * ROWS_W

    def _compute(buf):
        @plsc.parallel_loop(0, CHUNK, 4 * L, unroll=2)
        def _body(v):
            r = lax.shift_right_logical(v, COLS_LOG2)
            col = jnp.bitwise_and(v, COLS - 1)
            for k in range(4):
                ck = col + (k * L)
                xv = buf[r, pl.ds(ck, L)]
                t = xv * INV_H + OFF
                t = jnp.minimum(jnp.maximum(t, 0.0), float(NSEG - 1))
                idx = t.astype(jnp.int32)
                sv = plsc.load_gather(tab_s, [idx])
                av = plsc.load_gather(tab_a, [idx])
                buf[r, pl.ds(ck, L)] = av + sv * xv

    in_cp = [None] * NBUF
    out_cp = [None] * NBUF
    PF = NBUF - 1
    for b in range(min(PF, NCHUNK)):
        in_cp[b] = pltpu.async_copy(
            x_hbm.at[pl.ds(base + b * CROWS, CROWS)], ring.at[b], isems.at[b]
        )
    for c in range(NCHUNK):
        b = c % NBUF
        in_cp[b].wait()
        _compute(ring.at[b])
        out_cp[b] = pltpu.async_copy(
            ring.at[b], out_hbm.at[pl.ds(base + c * CROWS, CROWS)], osems.at[b]
        )
        if c + PF < NCHUNK:
            bn = (c + PF) % NBUF
            if c >= 1:
                out_cp[bn].wait()
            in_cp[bn] = pltpu.async_copy(
                x_hbm.at[pl.ds(base + (c + PF) * CROWS, CROWS)],
                ring.at[bn], isems.at[bn]
            )
    for c in range(max(0, NCHUNK - NBUF), NCHUNK):
        out_cp[c % NBUF].wait()


def kernel(x, act_array, xgrid):
    return _interp_sc(x, act_array, xgrid)


# speedup vs baseline: 1.1528x; 1.1528x over previous
import functools

import jax
import jax.numpy as jnp
from jax import lax
from jax.experimental import pallas as pl
from jax.experimental.pallas import tpu as pltpu
from jax.experimental.pallas import tpu_sc as plsc

N_GRID = 50
NSEG = N_GRID - 1
TAB = 64
NC, NS, L = 2, 16, 16
NW = NC * NS

ROWS, COLS = 4096, 2048
ROWS_W = ROWS // NW
CROWS = 16
CHUNK = CROWS * COLS
NCHUNK = ROWS_W // CROWS
RVECS = COLS // L
COLS_LOG2 = 11
NBUF = 3

INV_H = NSEG / 10.0
OFF = 5.0 * INV_H

_mesh = plsc.VectorSubcoreMesh(
    core_axis_name="c", subcore_axis_name="s", num_cores=NC, num_subcores=NS
)


@functools.partial(
    pl.kernel,
    out_type=jax.ShapeDtypeStruct((ROWS, COLS), jnp.float32),
    mesh=_mesh,
    compiler_params=pltpu.CompilerParams(needs_layout_passes=False),
    scratch_types=[
        pltpu.VMEM((N_GRID,), jnp.float32),
        pltpu.VMEM((N_GRID,), jnp.float32),
        pltpu.VMEM((TAB,), jnp.float32),
        pltpu.VMEM((TAB,), jnp.float32),
        pltpu.VMEM((NBUF, CROWS, COLS), jnp.float32),
        pltpu.SemaphoreType.DMA((NBUF,)),
        pltpu.SemaphoreType.DMA((NBUF,)),
    ],
)
def _interp_sc(x_hbm, y_hbm, g_hbm, out_hbm, ybuf, gbuf, tab_s, tab_a,
               ring, isems, osems):
    wid = lax.axis_index("s") * NC + lax.axis_index("c")

    pltpu.sync_copy(y_hbm, ybuf)
    pltpu.sync_copy(g_hbm, gbuf)

    lanes = lax.iota(jnp.int32, L)
    for j in range(TAB // L):
        k0 = jnp.minimum(lanes + (j * L), NSEG - 1)
        k1 = k0 + 1
        y0 = plsc.load_gather(ybuf, [k0])
        y1 = plsc.load_gather(ybuf, [k1])
        g0 = plsc.load_gather(gbuf, [k0])
        g1 = plsc.load_gather(gbuf, [k1])
        s = (y1 - y0) / (g1 - g0)
        tab_s[pl.ds(j * L, L)] = s
        tab_a[pl.ds(j * L, L)] = y0 - s * g0

    base = wid * ROWS_W

    def _compute(buf):
        @plsc.parallel_loop(0, CHUNK, L, unroll=8)
        def _body(v):
            r = lax.shift_right_logical(v, COLS_LOG2)
            col = jnp.bitwise_and(v, COLS - 1)
            xv = buf[r, pl.ds(col, L)]
            t = xv * INV_H + OFF
            t = jnp.minimum(jnp.maximum(t, 0.0), float(NSEG - 1))
            idx = t.astype(jnp.int32)
            sv = plsc.load_gather(tab_s, [idx])
            av = plsc.load_gather(tab_a, [idx])
            buf[r, pl.ds(col, L)] = av + sv * xv

    in_cp = [None] * NBUF
    out_cp = [None] * NBUF
    PF = NBUF - 1
    for b in range(min(PF, NCHUNK)):
        in_cp[b] = pltpu.async_copy(
            x_hbm.at[pl.ds(base + b * CROWS, CROWS)], ring.at[b], isems.at[b]
        )
    for c in range(NCHUNK):
        b = c % NBUF
        in_cp[b].wait()
        _compute(ring.at[b])
        out_cp[b] = pltpu.async_copy(
            ring.at[b], out_hbm.at[pl.ds(base + c * CROWS, CROWS)], osems.at[b]
        )
        if c + PF < NCHUNK:
            bn = (c + PF) % NBUF
            if c >= 1:
                out_cp[bn].wait()
            in_cp[bn] = pltpu.async_copy(
                x_hbm.at[pl.ds(base + (c + PF) * CROWS, CROWS)],
                ring.at[bn], isems.at[bn]
            )
    for c in range(max(0, NCHUNK - NBUF), NCHUNK):
        out_cp[c % NBUF].wait()


def kernel(x, act_array, xgrid):
    return _interp_sc(x, act_array, xgrid)


# speedup vs baseline: 1.2043x; 1.0447x over previous
import functools

import jax
import jax.numpy as jnp
from jax import lax
from jax.experimental import pallas as pl
from jax.experimental.pallas import tpu as pltpu
from jax.experimental.pallas import tpu_sc as plsc

N_GRID = 50
NSEG = N_GRID - 1
TAB = 64
NC, NS, L = 2, 16, 16
NW = NC * NS

ROWS, COLS = 4096, 2048
ROWS_W = ROWS // NW
CROWS = 16
CHUNK = CROWS * COLS
NCHUNK = ROWS_W // CROWS
COLS_LOG2 = 11
NBUF = 3

INV_H = NSEG / 10.0
OFF = 5.0 * INV_H

_mesh = plsc.VectorSubcoreMesh(
    core_axis_name="c", subcore_axis_name="s", num_cores=NC, num_subcores=NS
)


@functools.partial(
    pl.kernel,
    out_type=jax.ShapeDtypeStruct((ROWS, COLS), jnp.float32),
    mesh=_mesh,
    compiler_params=pltpu.CompilerParams(needs_layout_passes=False),
    scratch_types=[
        pltpu.VMEM((N_GRID,), jnp.float32),
        pltpu.VMEM((N_GRID,), jnp.float32),
        pltpu.VMEM((TAB,), jnp.float32),
        pltpu.VMEM((TAB,), jnp.float32),
        pltpu.VMEM((NBUF, CROWS, COLS), jnp.float32),
        pltpu.SemaphoreType.DMA((NBUF,)),
        pltpu.SemaphoreType.DMA((NBUF,)),
    ],
)
def _interp_sc(x_hbm, y_hbm, g_hbm, out_hbm, ybuf, gbuf, tab_s, tab_a,
               ring, isems, osems):
    wid = lax.axis_index("s") * NC + lax.axis_index("c")
    base = wid * ROWS_W

    in_cp = [None] * NBUF
    out_cp = [None] * NBUF
    PF = NBUF - 1
    for b in range(min(PF, NCHUNK)):
        in_cp[b] = pltpu.async_copy(
            x_hbm.at[pl.ds(base + b * CROWS, CROWS)], ring.at[b], isems.at[b]
        )

    pltpu.sync_copy(y_hbm, ybuf)
    pltpu.sync_copy(g_hbm, gbuf)

    lanes = lax.iota(jnp.int32, L)
    for j in range(TAB // L):
        k0 = jnp.minimum(lanes + (j * L), NSEG - 1)
        k1 = k0 + 1
        y0 = plsc.load_gather(ybuf, [k0])
        y1 = plsc.load_gather(ybuf, [k1])
        g0 = plsc.load_gather(gbuf, [k0])
        g1 = plsc.load_gather(gbuf, [k1])
        s = (y1 - y0) / (g1 - g0)
        tab_s[pl.ds(j * L, L)] = s
        tab_a[pl.ds(j * L, L)] = y0 - s * g0

    def _compute(buf):
        @plsc.parallel_loop(0, CHUNK, L, unroll=8)
        def _body(v):
            r = lax.shift_right_logical(v, COLS_LOG2)
            col = jnp.bitwise_and(v, COLS - 1)
            xv = buf[r, pl.ds(col, L)]
            t = xv * INV_H + OFF
            t = jnp.minimum(jnp.maximum(t, 0.0), float(NSEG - 1))
            idx = t.astype(jnp.int32)
            sv = plsc.load_gather(tab_s, [idx])
            av = plsc.load_gather(tab_a, [idx])
            buf[r, pl.ds(col, L)] = av + sv * xv

    for c in range(NCHUNK):
        b = c % NBUF
        in_cp[b].wait()
        _compute(ring.at[b])
        out_cp[b] = pltpu.async_copy(
            ring.at[b], out_hbm.at[pl.ds(base + c * CROWS, CROWS)], osems.at[b]
        )
        if c + PF < NCHUNK:
            bn = (c + PF) % NBUF
            if c >= 1:
                out_cp[bn].wait()
            in_cp[bn] = pltpu.async_copy(
                x_hbm.at[pl.ds(base + (c + PF) * CROWS, CROWS)],
                ring.at[bn], isems.at[bn]
            )
    for c in range(max(0, NCHUNK - NBUF), NCHUNK):
        out_cp[c % NBUF].wait()


def kernel(x, act_array, xgrid):
    return _interp_sc(x, act_array, xgrid)
